# initial kernel scaffold (unmeasured)
import jax
import jax.numpy as jnp
from jax import lax
from jax.experimental import pallas as pl
from jax.experimental.pallas import tpu as pltpu

T = 2048
D = 4096
V_LOCAL = 8192
TB = T // 2
VB = 1024
K = V_LOCAL // VB


def kernel(x, W, labels):
    my_x = lax.axis_index("x")
    x_blk = lax.dynamic_slice_in_dim(x, my_x * TB, TB, axis=0)
    lab_blk = lax.dynamic_slice_in_dim(labels, my_x * TB, TB, axis=0)
    lab_blk = lab_blk.reshape(TB, 1)

    def body(x_ref, w_ref, lab_ref, out_ref,
             acc_ref, recv_y_ref, nll_ref, recv_x_ref, sems):
        j = pl.program_id(0)
        mx = lax.axis_index("x")
        my = lax.axis_index("y")

        @pl.when(j == 0)
        def _():
            acc_ref[...] = jnp.zeros_like(acc_ref)

        logits = jnp.dot(x_ref[...], w_ref[...],
                         preferred_element_type=jnp.float32)
        s = jnp.sum(jnp.exp(logits), axis=1, keepdims=True)
        ids = (my * V_LOCAL + j * VB
               + lax.broadcasted_iota(jnp.int32, (TB, VB), 1))
        lab = jnp.sum(jnp.where(ids == lab_ref[...], logits, 0.0),
                      axis=1, keepdims=True)
        acc_ref[...] = acc_ref[...] + jnp.concatenate([s, lab], axis=1)

        @pl.when(j == K - 1)
        def _():
            rdma_y = pltpu.make_async_remote_copy(
                src_ref=acc_ref,
                dst_ref=recv_y_ref,
                send_sem=sems.at[0],
                recv_sem=sems.at[1],
                device_id=(mx, 1 - my),
                device_id_type=pl.DeviceIdType.MESH,
            )
            rdma_y.start()
            rdma_y.wait()
            tot = acc_ref[...] + recv_y_ref[...]
            nll = jnp.log(tot[:, 0:1]) - tot[:, 1:2]
            nll_ref[...] = nll
            out_ref[pl.ds(mx * TB, TB), :] = nll

            rdma_x = pltpu.make_async_remote_copy(
                src_ref=nll_ref,
                dst_ref=recv_x_ref,
                send_sem=sems.at[2],
                recv_sem=sems.at[3],
                device_id=(1 - mx, my),
                device_id_type=pl.DeviceIdType.MESH,
            )
            rdma_x.start()
            rdma_x.wait()
            out_ref[pl.ds((1 - mx) * TB, TB), :] = recv_x_ref[...]

    out = pl.pallas_call(
        body,
        grid=(K,),
        in_specs=[
            pl.BlockSpec((TB, D), lambda j: (0, 0)),
            pl.BlockSpec((D, VB), lambda j: (0, j)),
            pl.BlockSpec((TB, 1), lambda j: (0, 0)),
        ],
        out_specs=pl.BlockSpec((T, 1), lambda j: (0, 0)),
        out_shape=jax.ShapeDtypeStruct((T, 1), jnp.float32),
        scratch_shapes=[
            pltpu.VMEM((TB, 2), jnp.float32),
            pltpu.VMEM((TB, 2), jnp.float32),
            pltpu.VMEM((TB, 1), jnp.float32),
            pltpu.VMEM((TB, 1), jnp.float32),
            pltpu.SemaphoreType.DMA((4,)),
        ],
        compiler_params=pltpu.CompilerParams(
            dimension_semantics=("arbitrary",),
            collective_id=0,
        ),
    )(x_blk, W, lab_blk)
    return out.reshape(T)


# baseline (device time: 125465 ns/iter reference)
import jax
import jax.numpy as jnp
from jax import lax
from jax.experimental import pallas as pl
from jax.experimental.pallas import tpu as pltpu

T = 2048
D = 4096
V_LOCAL = 8192
TB = T // 2
VB = 1024
K = V_LOCAL // VB


def kernel(x, W, labels):
    my_x = lax.axis_index("x")
    x_blk = lax.dynamic_slice_in_dim(x, my_x * TB, TB, axis=0)
    lab_blk = lax.dynamic_slice_in_dim(labels, my_x * TB, TB, axis=0)
    lab_blk = lab_blk.reshape(TB, 1)

    def body(x_ref, w_ref, lab_ref, out_ref,
             acc_ref, recv_y_ref, nll_ref, recv_x_ref, sems):
        j = pl.program_id(0)
        mx = lax.axis_index("x")
        my = lax.axis_index("y")

        @pl.when(j == 0)
        def _():
            acc_ref[...] = jnp.zeros_like(acc_ref)

        logits = jnp.dot(x_ref[...], w_ref[...],
                         preferred_element_type=jnp.float32)
        s = jnp.sum(jnp.exp(logits), axis=1, keepdims=True)
        ids = (my * V_LOCAL + j * VB
               + lax.broadcasted_iota(jnp.int32, (TB, VB), 1))
        lab = jnp.sum(jnp.where(ids == lab_ref[...], logits, 0.0),
                      axis=1, keepdims=True)
        acc_ref[...] = acc_ref[...] + jnp.concatenate([s, lab], axis=1)

        @pl.when(j == K - 1)
        def _():
            rdma_y = pltpu.make_async_remote_copy(
                src_ref=acc_ref,
                dst_ref=recv_y_ref,
                send_sem=sems.at[0],
                recv_sem=sems.at[1],
                device_id=(mx, 1 - my),
                device_id_type=pl.DeviceIdType.MESH,
            )
            rdma_y.start()
            rdma_y.wait()
            tot = acc_ref[...] + recv_y_ref[...]
            nll = jnp.log(tot[:, 0:1]) - tot[:, 1:2]
            nll_ref[...] = nll
            out_ref[pl.ds(mx * TB, TB), :] = nll

            rdma_x = pltpu.make_async_remote_copy(
                src_ref=nll_ref,
                dst_ref=recv_x_ref,
                send_sem=sems.at[2],
                recv_sem=sems.at[3],
                device_id=(1 - mx, my),
                device_id_type=pl.DeviceIdType.MESH,
            )
            rdma_x.start()
            rdma_x.wait()
            out_ref[pl.ds((1 - mx) * TB, TB), :] = recv_x_ref[...]

    out = pl.pallas_call(
        body,
        grid=(K,),
        in_specs=[
            pl.BlockSpec((TB, D), lambda j: (0, 0)),
            pl.BlockSpec((D, VB), lambda j: (0, j)),
            pl.BlockSpec((TB, 1), lambda j: (0, 0)),
        ],
        out_specs=pl.BlockSpec((T, 1), lambda j: (0, 0)),
        out_shape=jax.ShapeDtypeStruct((T, 1), jnp.float32),
        scratch_shapes=[
            pltpu.VMEM((TB, 2), jnp.float32),
            pltpu.VMEM((TB, 2), jnp.float32),
            pltpu.VMEM((TB, 1), jnp.float32),
            pltpu.VMEM((TB, 1), jnp.float32),
            pltpu.SemaphoreType.DMA((4,)),
        ],
        compiler_params=pltpu.CompilerParams(
            dimension_semantics=("arbitrary",),
            vmem_limit_bytes=100 * 1024 * 1024,
        ),
    )(x_blk, W, lab_blk)
    return out.reshape(T)


# device time: 113188 ns/iter; 1.1085x vs baseline; 1.1085x over previous
import jax
import jax.numpy as jnp
from jax import lax
from jax.experimental import pallas as pl
from jax.experimental.pallas import tpu as pltpu

T = 2048
D = 4096
V_LOCAL = 8192
TB = T // 2
VB = 1024
K = V_LOCAL // VB


def kernel(x, W, labels):
    mx_arr = lax.axis_index("x").astype(jnp.int32).reshape(1)
    labels2d = labels.reshape(T, 1)

    def body(mx_ref, x_ref, w_ref, lab_ref, out_ref,
             acc_ref, recv_y_ref, nll_ref, recv_x_ref, sems):
        j = pl.program_id(0)
        mx = lax.axis_index("x")
        my = lax.axis_index("y")

        @pl.when(j == 0)
        def _():
            acc_ref[...] = jnp.zeros_like(acc_ref)

        logits = jnp.dot(x_ref[...], w_ref[...],
                         preferred_element_type=jnp.float32)
        s = jnp.sum(jnp.exp(logits), axis=1, keepdims=True)
        ids = (my * V_LOCAL + j * VB
               + lax.broadcasted_iota(jnp.int32, (TB, VB), 1))
        lab = jnp.sum(jnp.where(ids == lab_ref[...], logits, 0.0),
                      axis=1, keepdims=True)
        acc_ref[...] = acc_ref[...] + jnp.concatenate([s, lab], axis=1)

        @pl.when(j == K - 1)
        def _():
            rdma_y = pltpu.make_async_remote_copy(
                src_ref=acc_ref,
                dst_ref=recv_y_ref,
                send_sem=sems.at[0],
                recv_sem=sems.at[1],
                device_id=(mx, 1 - my),
                device_id_type=pl.DeviceIdType.MESH,
            )
            rdma_y.start()
            rdma_y.wait()
            tot = acc_ref[...] + recv_y_ref[...]
            nll = jnp.log(tot[:, 0:1]) - tot[:, 1:2]
            nll_ref[...] = nll
            out_ref[pl.ds(mx * TB, TB), :] = nll

            rdma_x = pltpu.make_async_remote_copy(
                src_ref=nll_ref,
                dst_ref=recv_x_ref,
                send_sem=sems.at[2],
                recv_sem=sems.at[3],
                device_id=(1 - mx, my),
                device_id_type=pl.DeviceIdType.MESH,
            )
            rdma_x.start()
            rdma_x.wait()
            out_ref[pl.ds((1 - mx) * TB, TB), :] = recv_x_ref[...]

    grid_spec = pltpu.PrefetchScalarGridSpec(
        num_scalar_prefetch=1,
        grid=(K,),
        in_specs=[
            pl.BlockSpec((TB, D), lambda j, mx: (mx[0], 0)),
            pl.BlockSpec((D, VB), lambda j, mx: (0, j)),
            pl.BlockSpec((TB, 1), lambda j, mx: (mx[0], 0)),
        ],
        out_specs=pl.BlockSpec((T, 1), lambda j, mx: (0, 0)),
        scratch_shapes=[
            pltpu.VMEM((TB, 2), jnp.float32),
            pltpu.VMEM((TB, 2), jnp.float32),
            pltpu.VMEM((TB, 1), jnp.float32),
            pltpu.VMEM((TB, 1), jnp.float32),
            pltpu.SemaphoreType.DMA((4,)),
        ],
    )

    out = pl.pallas_call(
        body,
        grid_spec=grid_spec,
        out_shape=jax.ShapeDtypeStruct((T, 1), jnp.float32),
        compiler_params=pltpu.CompilerParams(
            dimension_semantics=("arbitrary",),
            vmem_limit_bytes=100 * 1024 * 1024,
        ),
    )(mx_arr, x, W, labels2d)
    return out.reshape(T)
